# Initial kernel scaffold; baseline (speedup 1.0000x reference)
import jax
import jax.numpy as jnp
from jax.experimental import pallas as pl


def kernel(daytime, W_day, W_time, W_node):
    raise NotImplementedError("write your pallas kernel here")


# speedup vs baseline: 1.2316x; 1.2316x over previous
import functools

import jax
import jax.numpy as jnp
from jax import lax
from jax.experimental import pallas as pl
from jax.experimental.pallas import tpu as pltpu
from jax.experimental.pallas import tpu_sc as plsc

_NODE_COUNT = 325
_NODE_SIZE = 64
_DAY_SIZE = 32
_TIME_SIZE = 32
_ROW = _NODE_SIZE + _DAY_SIZE + _TIME_SIZE
_LANES = 16


@functools.lru_cache(maxsize=None)
def _make_sc_kernel(num_pairs):
    info = plsc.get_sparse_core_info()
    nc, ns = info.num_cores, info.num_subcores
    nw = nc * ns
    ppw = num_pairs // nw

    mesh = plsc.VectorSubcoreMesh(core_axis_name="c", subcore_axis_name="s")

    @functools.partial(
        pl.kernel,
        mesh=mesh,
        out_type=jax.ShapeDtypeStruct((num_pairs, _NODE_COUNT, _ROW), jnp.float32),
        scratch_types=[
            pltpu.VMEM((ppw,), jnp.int32),
            pltpu.VMEM((ppw,), jnp.int32),
            pltpu.VMEM((ppw, _ROW), jnp.float32),
            pltpu.VMEM((ppw, _ROW), jnp.float32),
            pltpu.VMEM((_NODE_COUNT, _NODE_SIZE), jnp.float32),
            pltpu.VMEM((_NODE_COUNT, _ROW), jnp.float32),
            pltpu.SemaphoreType.DMA,
        ],
    )
    def sc_embed(day_idx_hbm, time_idx_hbm, w_day_hbm, w_time_hbm, w_node_hbm,
                 out_hbm, didx_v, tidx_v, drows_v, trows_v, node_v, buf_v, sem):
        wid = lax.axis_index("s") * nc + lax.axis_index("c")
        base = wid * ppw

        pltpu.sync_copy(day_idx_hbm.at[pl.ds(base, ppw)], didx_v)
        pltpu.sync_copy(time_idx_hbm.at[pl.ds(base, ppw)], tidx_v)
        pltpu.async_copy(w_day_hbm.at[didx_v], drows_v, sem).wait()
        pltpu.async_copy(w_time_hbm.at[tidx_v], trows_v, sem).wait()
        pltpu.sync_copy(w_node_hbm, node_v)

        def init_row(r, carry):
            for c in range(_NODE_SIZE // _LANES):
                buf_v[r, pl.ds(_LANES * c, _LANES)] = node_v[r, pl.ds(_LANES * c, _LANES)]
            return carry

        lax.fori_loop(0, _NODE_COUNT, init_row, 0)

        def do_pair(j, carry):
            d0 = drows_v[j, pl.ds(0, _LANES)]
            d1 = drows_v[j, pl.ds(_LANES, _LANES)]
            t0 = trows_v[j, pl.ds(0, _LANES)]
            t1 = trows_v[j, pl.ds(_LANES, _LANES)]

            def brow(r, inner):
                buf_v[r, pl.ds(_NODE_SIZE, _LANES)] = d0
                buf_v[r, pl.ds(_NODE_SIZE + _LANES, _LANES)] = d1
                buf_v[r, pl.ds(_NODE_SIZE + 2 * _LANES, _LANES)] = t0
                buf_v[r, pl.ds(_NODE_SIZE + 3 * _LANES, _LANES)] = t1
                return inner

            lax.fori_loop(0, _NODE_COUNT, brow, 0)
            pltpu.sync_copy(buf_v, out_hbm.at[base + j])
            return carry

        lax.fori_loop(0, ppw, do_pair, 0)

    return sc_embed


def kernel(daytime, W_day, W_time, W_node):
    batch, len_seq, _ = daytime.shape
    flat = daytime.reshape(batch * len_seq, 2)
    day_idx = flat[:, 0].astype(jnp.int32)
    time_idx = flat[:, 1].astype(jnp.int32)
    w_day_p = jnp.pad(W_day, ((0, 0), (0, _ROW - W_day.shape[1])))
    w_time_p = jnp.pad(W_time, ((0, 0), (0, _ROW - W_time.shape[1])))
    sc = _make_sc_kernel(batch * len_seq)
    out = sc(day_idx, time_idx, w_day_p, w_time_p, W_node)
    return out.reshape(batch, len_seq, _NODE_COUNT, _ROW)


# speedup vs baseline: 1.3359x; 1.0847x over previous
import functools

import jax
import jax.numpy as jnp
from jax import lax
from jax.experimental import pallas as pl
from jax.experimental.pallas import tpu as pltpu
from jax.experimental.pallas import tpu_sc as plsc

_NODE_COUNT = 325
_NODE_SIZE = 64
_DAY_SIZE = 32
_TIME_SIZE = 32
_ROW = _NODE_SIZE + _DAY_SIZE + _TIME_SIZE
_LANES = 16
_RUNROLL = 5


@functools.lru_cache(maxsize=None)
def _make_sc_kernel(num_pairs):
    info = plsc.get_sparse_core_info()
    nc, ns = info.num_cores, info.num_subcores
    nw = nc * ns
    ppw = num_pairs // nw

    mesh = plsc.VectorSubcoreMesh(core_axis_name="c", subcore_axis_name="s")

    @functools.partial(
        pl.kernel,
        mesh=mesh,
        out_type=jax.ShapeDtypeStruct((num_pairs, _NODE_COUNT, _ROW), jnp.float32),
        scratch_types=[
            pltpu.VMEM((ppw,), jnp.int32),
            pltpu.VMEM((ppw,), jnp.int32),
            pltpu.VMEM((ppw, _ROW), jnp.float32),
            pltpu.VMEM((ppw, _ROW), jnp.float32),
            pltpu.VMEM((_NODE_COUNT, _ROW), jnp.float32),
            pltpu.VMEM((_NODE_COUNT, _ROW), jnp.float32),
            pltpu.SemaphoreType.DMA,
            pltpu.SemaphoreType.DMA,
        ],
    )
    def sc_embed(day_idx_hbm, time_idx_hbm, w_day_hbm, w_time_hbm, w_node_hbm,
                 out_hbm, didx_v, tidx_v, drows_v, trows_v,
                 buf0_v, buf1_v, sem0, sem1):
        wid = lax.axis_index("s") * nc + lax.axis_index("c")
        base = wid * ppw
        bufs = (buf0_v, buf1_v)
        sems = (sem0, sem1)

        pltpu.sync_copy(day_idx_hbm.at[pl.ds(base, ppw)], didx_v)
        pltpu.sync_copy(time_idx_hbm.at[pl.ds(base, ppw)], tidx_v)
        pltpu.async_copy(w_day_hbm.at[didx_v], drows_v, sem0).wait()
        pltpu.async_copy(w_time_hbm.at[tidx_v], trows_v, sem0).wait()

        pltpu.sync_copy(w_node_hbm, buf0_v)
        pltpu.sync_copy(w_node_hbm, buf1_v)

        def build(j, buf):
            d0 = drows_v[j, pl.ds(0, _LANES)]
            d1 = drows_v[j, pl.ds(_LANES, _LANES)]
            t0 = trows_v[j, pl.ds(0, _LANES)]
            t1 = trows_v[j, pl.ds(_LANES, _LANES)]

            def brow(r, inner):
                for k in range(_RUNROLL):
                    buf[_RUNROLL * r + k, pl.ds(_NODE_SIZE, _LANES)] = d0
                    buf[_RUNROLL * r + k, pl.ds(_NODE_SIZE + _LANES, _LANES)] = d1
                    buf[_RUNROLL * r + k, pl.ds(_NODE_SIZE + 2 * _LANES, _LANES)] = t0
                    buf[_RUNROLL * r + k, pl.ds(_NODE_SIZE + 3 * _LANES, _LANES)] = t1
                return inner

            lax.fori_loop(0, _NODE_COUNT // _RUNROLL, brow, 0)

        for b in range(2):
            build(b, bufs[b])
            pltpu.async_copy(bufs[b], out_hbm.at[base + b], sems[b])

        def step(g, carry):
            for b in range(2):
                j = 2 * g + b
                pltpu.make_async_copy(out_hbm.at[base], bufs[b], sems[b]).wait()
                build(j, bufs[b])
                pltpu.async_copy(bufs[b], out_hbm.at[base + j], sems[b])
            return carry

        lax.fori_loop(1, ppw // 2, step, 0)

        for b in range(2):
            pltpu.make_async_copy(out_hbm.at[base], bufs[b], sems[b]).wait()

    return sc_embed


def kernel(daytime, W_day, W_time, W_node):
    batch, len_seq, _ = daytime.shape
    flat = daytime.reshape(batch * len_seq, 2)
    day_idx = flat[:, 0].astype(jnp.int32)
    time_idx = flat[:, 1].astype(jnp.int32)
    w_day_p = jnp.pad(W_day, ((0, 0), (0, _ROW - W_day.shape[1])))
    w_time_p = jnp.pad(W_time, ((0, 0), (0, _ROW - W_time.shape[1])))
    w_node_p = jnp.pad(W_node, ((0, 0), (0, _ROW - W_node.shape[1])))
    sc = _make_sc_kernel(batch * len_seq)
    out = sc(day_idx, time_idx, w_day_p, w_time_p, w_node_p)
    return out.reshape(batch, len_seq, _NODE_COUNT, _ROW)


# speedup vs baseline: 1.4728x; 1.1026x over previous
import functools

import jax
import jax.numpy as jnp
from jax import lax
from jax.experimental import pallas as pl
from jax.experimental.pallas import tpu as pltpu
from jax.experimental.pallas import tpu_sc as plsc

_NODE_COUNT = 325
_NODE_SIZE = 64
_DAY_SIZE = 32
_TIME_SIZE = 32
_ROW = _NODE_SIZE + _DAY_SIZE + _TIME_SIZE
_LANES = 16
_NCHUNKS = 8
_CHUNK = (_NODE_COUNT + _NCHUNKS - 1) // _NCHUNKS


@functools.lru_cache(maxsize=None)
def _make_sc_kernel(batch, len_seq):
    info = plsc.get_sparse_core_info()
    nc, ns = info.num_cores, info.num_subcores
    nw = nc * ns
    items_per_worker = (len_seq * _NCHUNKS) // nw

    mesh = plsc.VectorSubcoreMesh(core_axis_name="c", subcore_axis_name="s")

    @functools.partial(
        pl.kernel,
        mesh=mesh,
        out_type=jax.ShapeDtypeStruct(
            (len_seq, _NODE_COUNT, batch, _ROW), jnp.float32),
        scratch_types=[
            pltpu.VMEM((batch,), jnp.int32),
            pltpu.VMEM((batch,), jnp.int32),
            pltpu.VMEM((batch, _ROW), jnp.float32),
            pltpu.VMEM((batch, _ROW), jnp.float32),
            pltpu.VMEM((_NODE_COUNT, _ROW), jnp.float32),
            pltpu.VMEM((batch, _ROW), jnp.float32),
            pltpu.VMEM((batch, _ROW), jnp.float32),
            pltpu.SemaphoreType.DMA,
            pltpu.SemaphoreType.DMA,
            pltpu.SemaphoreType.DMA,
        ],
    )
    def sc_embed(didx_hbm, tidx_hbm, w_day_hbm, w_time_hbm, w_node_hbm,
                 out_hbm, didx_v, tidx_v, drows_v, trows_v, node_v,
                 buf0_v, buf1_v, sem0, sem1, gsem):
        wid = lax.axis_index("s") * nc + lax.axis_index("c")

        pltpu.sync_copy(w_node_hbm, node_v)

        def drain(buf, sem):
            pltpu.make_async_copy(out_hbm.at[0, 0], buf, sem).wait()

        def build_node(n, buf):
            v0 = node_v[n, pl.ds(0, _LANES)]
            v1 = node_v[n, pl.ds(_LANES, _LANES)]
            v2 = node_v[n, pl.ds(2 * _LANES, _LANES)]
            v3 = node_v[n, pl.ds(3 * _LANES, _LANES)]

            def nrow(r, inner):
                for k in range(4):
                    b = 4 * r + k
                    buf[b, pl.ds(0, _LANES)] = v0
                    buf[b, pl.ds(_LANES, _LANES)] = v1
                    buf[b, pl.ds(2 * _LANES, _LANES)] = v2
                    buf[b, pl.ds(3 * _LANES, _LANES)] = v3
                return inner

            lax.fori_loop(0, batch // 4, nrow, 0)

        for q in range(items_per_worker):
            item = wid + nw * q
            l = lax.shift_right_logical(item, 3)
            ch = lax.bitwise_and(item, _NCHUNKS - 1)
            n0 = ch * _CHUNK
            cnt = jnp.minimum(_NODE_COUNT - n0, _CHUNK)

            pltpu.sync_copy(didx_hbm.at[l], didx_v)
            pltpu.sync_copy(tidx_hbm.at[l], tidx_v)
            pltpu.async_copy(w_day_hbm.at[didx_v], drows_v, gsem).wait()
            pltpu.async_copy(w_time_hbm.at[tidx_v], trows_v, gsem).wait()

            def dtrow(b, inner):
                d0 = drows_v[b, pl.ds(0, _LANES)]
                d1 = drows_v[b, pl.ds(_LANES, _LANES)]
                t0 = trows_v[b, pl.ds(0, _LANES)]
                t1 = trows_v[b, pl.ds(_LANES, _LANES)]
                for buf in (buf0_v, buf1_v):
                    buf[b, pl.ds(_NODE_SIZE, _LANES)] = d0
                    buf[b, pl.ds(_NODE_SIZE + _LANES, _LANES)] = d1
                    buf[b, pl.ds(_NODE_SIZE + 2 * _LANES, _LANES)] = t0
                    buf[b, pl.ds(_NODE_SIZE + 3 * _LANES, _LANES)] = t1
                return inner

            lax.fori_loop(0, batch, dtrow, 0)

            build_node(n0, buf0_v)
            pltpu.async_copy(buf0_v, out_hbm.at[l, n0], sem0)
            build_node(n0 + 1, buf1_v)
            pltpu.async_copy(buf1_v, out_hbm.at[l, n0 + 1], sem1)

            def step(g, carry):
                n = n0 + 2 * g
                drain(buf0_v, sem0)
                build_node(n, buf0_v)
                pltpu.async_copy(buf0_v, out_hbm.at[l, n], sem0)
                drain(buf1_v, sem1)
                build_node(n + 1, buf1_v)
                pltpu.async_copy(buf1_v, out_hbm.at[l, n + 1], sem1)
                return carry

            m = lax.shift_right_logical(cnt, 1)
            lax.fori_loop(1, m, step, 0)

            nt = n0 + cnt - 1
            drain(buf0_v, sem0)
            build_node(nt, buf0_v)
            pltpu.async_copy(buf0_v, out_hbm.at[l, nt], sem0)

            drain(buf0_v, sem0)
            drain(buf1_v, sem1)

    return sc_embed


def kernel(daytime, W_day, W_time, W_node):
    batch, len_seq, _ = daytime.shape
    didx = daytime[:, :, 0].T.astype(jnp.int32)
    tidx = daytime[:, :, 1].T.astype(jnp.int32)
    w_day_p = jnp.pad(W_day, ((0, 0), (0, _ROW - W_day.shape[1])))
    w_time_p = jnp.pad(W_time, ((0, 0), (0, _ROW - W_time.shape[1])))
    w_node_p = jnp.pad(W_node, ((0, 0), (0, _ROW - W_node.shape[1])))
    sc = _make_sc_kernel(batch, len_seq)
    out = sc(didx, tidx, w_day_p, w_time_p, w_node_p)
    return out.transpose(2, 0, 1, 3)
